# Initial kernel scaffold; baseline (speedup 1.0000x reference)
import jax
import jax.numpy as jnp
from jax.experimental import pallas as pl


def kernel(x, W_flr, b_flr, W_s, b_s, W_out, b_out):
    raise NotImplementedError("write your pallas kernel here")


# speedup vs baseline: 14.2737x; 14.2737x over previous
import jax
import jax.numpy as jnp
from jax.experimental import pallas as pl
from jax.experimental.pallas import tpu as pltpu

_B, _V, _F = 8, 2048, 64
_K = 16
_NDIM = 4
_NPROP = 64
_NFILT = 128
_EXPF = 10.0
_R = 256


def _xform_body(x_ref, Ws_ref, bs_ref, Wf_ref, bf_ref, c_ref, f_ref):
    xb = x_ref[0].astype(jnp.bfloat16)
    c_ref[0] = (jnp.dot(xb, Ws_ref[...].astype(jnp.bfloat16),
                        preferred_element_type=jnp.float32) + bs_ref[...])
    f_ref[0] = (jnp.dot(xb, Wf_ref[...].astype(jnp.bfloat16),
                        preferred_element_type=jnp.float32) + bf_ref[...])


def _main_body(x_rows_ref, c_rows_ref, c_all_ref, sqc_ref, sqr_ref,
               feats_ref, Wo_ref, bo_ref, out_ref):
    x_rows = x_rows_ref[0]
    c_rows = c_rows_ref[0]
    c_all = c_all_ref[0]
    sq_col = sqc_ref[0][:, 0:1]
    sq_row = sqr_ref[0][0:1, :]
    g = jax.lax.dot_general(c_rows.astype(jnp.bfloat16),
                            c_all.astype(jnp.bfloat16),
                            (((1,), (1,)), ((), ())),
                            preferred_element_type=jnp.float32)
    dist = jnp.abs((-2.0 * g + sq_col) + sq_row)
    negd = -dist
    features = feats_ref[0]
    iota = jax.lax.broadcasted_iota(jnp.int32, (_R, _V), 1)
    sumacc = jnp.zeros((_R, _NPROP), jnp.float32)
    maxacc = jnp.full((_R, _NPROP), -jnp.inf, jnp.float32)
    for k in range(_K):
        m = jnp.max(negd, axis=1, keepdims=True)
        sel = jnp.where(negd == m, iota, _V)
        idx = jnp.min(sel, axis=1, keepdims=True)
        onehot = iota == idx
        negd = jnp.where(onehot, -jnp.inf, negd)
        if k > 0:
            w = jnp.exp(_EXPF * m)
            g_k = jnp.dot(onehot.astype(jnp.float32), features,
                          preferred_element_type=jnp.float32)
            wk = w * g_k
            sumacc = sumacc + wk
            maxacc = jnp.maximum(maxacc, wk)
    mean = sumacc * (1.0 / (_K - 1))
    Wo = Wo_ref[...]
    out = (jnp.dot(x_rows.astype(jnp.bfloat16),
                   Wo[0:_F].astype(jnp.bfloat16),
                   preferred_element_type=jnp.float32)
           + jnp.dot(mean.astype(jnp.bfloat16),
                     Wo[_F:_F + _NPROP].astype(jnp.bfloat16),
                     preferred_element_type=jnp.float32)
           + jnp.dot(maxacc.astype(jnp.bfloat16),
                     Wo[_F + _NPROP:].astype(jnp.bfloat16),
                     preferred_element_type=jnp.float32)
           + bo_ref[...])
    out_ref[0] = out


def kernel(x, W_flr, b_flr, W_s, b_s, W_out, b_out):
    Ws_pad = jnp.zeros((_F, 128), jnp.float32).at[:, :_NDIM].set(W_s)
    bs_pad = jnp.zeros((1, 128), jnp.float32).at[:, :_NDIM].set(b_s)
    bflr = b_flr.reshape(1, _NPROP)
    bo = b_out.reshape(1, _NFILT)

    cpad, feats = pl.pallas_call(
        _xform_body,
        grid=(_B,),
        in_specs=[
            pl.BlockSpec((1, _V, _F), lambda b: (b, 0, 0)),
            pl.BlockSpec((_F, 128), lambda b: (0, 0)),
            pl.BlockSpec((1, 128), lambda b: (0, 0)),
            pl.BlockSpec((_F, _NPROP), lambda b: (0, 0)),
            pl.BlockSpec((1, _NPROP), lambda b: (0, 0)),
        ],
        out_specs=[
            pl.BlockSpec((1, _V, 128), lambda b: (b, 0, 0)),
            pl.BlockSpec((1, _V, _NPROP), lambda b: (b, 0, 0)),
        ],
        out_shape=[
            jax.ShapeDtypeStruct((_B, _V, 128), jnp.float32),
            jax.ShapeDtypeStruct((_B, _V, _NPROP), jnp.float32),
        ],
        compiler_params=pltpu.CompilerParams(
            dimension_semantics=("parallel",)),
    )(x, Ws_pad, bs_pad, W_flr, bflr)

    csl = cpad[:, :, :_NDIM]
    sq = jnp.sum(csl * csl, axis=2)
    sq_col = jnp.broadcast_to(sq[:, :, None], (_B, _V, 8))
    sq_row = jnp.broadcast_to(sq[:, None, :], (_B, 8, _V))

    return pl.pallas_call(
        _main_body,
        grid=(_B, _V // _R),
        in_specs=[
            pl.BlockSpec((1, _R, _F), lambda b, r: (b, r, 0)),
            pl.BlockSpec((1, _R, 128), lambda b, r: (b, r, 0)),
            pl.BlockSpec((1, _V, 128), lambda b, r: (b, 0, 0)),
            pl.BlockSpec((1, _R, 8), lambda b, r: (b, r, 0)),
            pl.BlockSpec((1, 8, _V), lambda b, r: (b, 0, 0)),
            pl.BlockSpec((1, _V, _NPROP), lambda b, r: (b, 0, 0)),
            pl.BlockSpec((_F + 2 * _NPROP, _NFILT), lambda b, r: (0, 0)),
            pl.BlockSpec((1, _NFILT), lambda b, r: (0, 0)),
        ],
        out_specs=pl.BlockSpec((1, _R, _NFILT), lambda b, r: (b, r, 0)),
        out_shape=jax.ShapeDtypeStruct((_B, _V, _NFILT), jnp.float32),
        compiler_params=pltpu.CompilerParams(
            dimension_semantics=("parallel", "arbitrary")),
    )(x, cpad, cpad, sq_col, sq_row, feats, W_out, bo)
